# baseline (speedup 1.0000x reference)
import functools

import jax
import jax.numpy as jnp
from jax import lax
from jax.experimental import pallas as pl
from jax.experimental.pallas import tpu as pltpu
from jax.experimental.pallas import tpu_sc as plsc

_N = 1048576
_VOCAB = 10
_D = 16
_NC = 1
_NS = 16
_NW = _NC * _NS
_CHUNK = _N // _NW
_LANES = 16
_VECS = _CHUNK // _LANES

_M3 = 0x71C71C7
_INNER = 7
_CHAINS = 4
_BLOCK = _INNER * _CHAINS * 2
_OUTER = _VECS // _BLOCK
_TAIL = _VECS - _OUTER * _BLOCK
_SEG1 = _OUTER // 2
_SPLIT = _SEG1 * _BLOCK * _LANES


def _build_partial_call():
  mesh = plsc.VectorSubcoreMesh(
      core_axis_name="c", subcore_axis_name="s", num_cores=_NC)

  @functools.partial(
      pl.kernel,
      mesh=mesh,
      compiler_params=pltpu.CompilerParams(needs_layout_passes=False),
      out_type=jax.ShapeDtypeStruct((_NW, _D), jnp.float32),
      scratch_types=[
          pltpu.VMEM((_CHUNK,), jnp.int32),
          pltpu.VMEM((_VOCAB, _D), jnp.float32),
          pltpu.VMEM((_D,), jnp.float32),
          pltpu.VMEM((_LANES * _LANES,), jnp.int32),
          pltpu.VMEM((_LANES * _LANES,), jnp.int32),
          pltpu.SemaphoreType.DMA,
          pltpu.SemaphoreType.DMA,
      ],
  )
  def digit_hist_partial(digits_hbm, table_hbm, out_hbm,
                         chunk_v, table_v, row_v, cnt_v, rep_v,
                         sem1, sem2):
    c = lax.axis_index("c")
    s = lax.axis_index("s")
    wid = c * _NS + s
    base = wid * _CHUNK

    cp1 = pltpu.async_copy(
        digits_hbm.at[pl.ds(base, _SPLIT)], chunk_v.at[pl.ds(0, _SPLIT)], sem1)
    cp2 = pltpu.async_copy(
        digits_hbm.at[pl.ds(base + _SPLIT, _CHUNK - _SPLIT)],
        chunk_v.at[pl.ds(_SPLIT, _CHUNK - _SPLIT)], sem2)
    pltpu.sync_copy(table_hbm, table_v)

    def inner_block(vbase, n, ef):
      e, f = ef
      accs = [jnp.zeros((_LANES,), jnp.int32) for _ in range(_CHAINS)]
      for a in range(_CHAINS):
        for j in range(n):
          vec = chunk_v[pl.ds((vbase + a * n + j) * _LANES, _LANES)]
          accs[a] = accs[a] + (jnp.int32(1) << (vec * 3))
      for a in range(_CHAINS):
        e = e + (accs[a] & _M3)
        f = f + ((accs[a] >> 3) & _M3)
      return e, f

    def mid_flush(ef, cnts):
      e, f = ef
      cnts = list(cnts)
      for k in range(5):
        cnts[2 * k] = cnts[2 * k] + ((e >> (6 * k)) & 63)
        cnts[2 * k + 1] = cnts[2 * k + 1] + ((f >> (6 * k)) & 63)
      return tuple(cnts)

    def outer_body(i, cnts):
      ef = (jnp.zeros((_LANES,), jnp.int32), jnp.zeros((_LANES,), jnp.int32))
      vbase = i * _BLOCK
      ef = inner_block(vbase, _INNER, ef)
      ef = inner_block(vbase + _INNER * _CHAINS, _INNER, ef)
      return mid_flush(ef, cnts)

    cnts0 = tuple(jnp.zeros((_LANES,), jnp.int32) for _ in range(_VOCAB))
    cp1.wait()
    cnts = lax.fori_loop(0, _SEG1, outer_body, cnts0)
    cp2.wait()
    cnts = lax.fori_loop(_SEG1, _OUTER, outer_body, cnts)
    ef = (jnp.zeros((_LANES,), jnp.int32), jnp.zeros((_LANES,), jnp.int32))
    vb = _OUTER * _BLOCK
    q = _TAIL // _CHAINS
    while q > 0:
      n = min(q, _INNER)
      ef = inner_block(vb, n, ef)
      vb += n * _CHAINS
      q -= n
    accs = mid_flush(ef, cnts)

    for d in range(_VOCAB):
      cnt_v[pl.ds(d * _LANES, _LANES)] = accs[d]
    lane = lax.iota(jnp.int32, _LANES)
    totals = jnp.zeros((_LANES,), jnp.int32)
    for l in range(_LANES):
      totals = totals + plsc.load_gather(cnt_v, [lane * _LANES + l])
    for l in range(_LANES):
      plsc.store_scatter(rep_v, [lane * _LANES + l], totals)

    partial = jnp.zeros((_D,), jnp.float32)
    for d in range(_VOCAB):
      splat = rep_v[pl.ds(d * _LANES, _LANES)]
      partial = partial + splat.astype(jnp.float32) * table_v[d, :]
    row_v[...] = partial * jnp.float32(1.0 / _N)
    pltpu.sync_copy(row_v, out_hbm.at[wid])

  return digit_hist_partial


def _build_tc_fold():
  def fold_body(parts_ref, out_ref):
    out_ref[...] = jnp.sum(parts_ref[...], axis=0, keepdims=True)

  return pl.pallas_call(
      fold_body,
      out_shape=jax.ShapeDtypeStruct((1, _D), jnp.float32),
  )


_partial_call = _build_partial_call()
_tc_fold = _build_tc_fold()


def kernel(digits, table):
  digits = digits.reshape(_N).astype(jnp.int32)
  table = table.astype(jnp.float32)
  partials = _partial_call(digits, table)
  return _tc_fold(partials).reshape(_D)


# speedup vs baseline: 1.0512x; 1.0512x over previous
import functools

import jax
import jax.numpy as jnp
from jax import lax
from jax.experimental import pallas as pl
from jax.experimental.pallas import tpu as pltpu
from jax.experimental.pallas import tpu_sc as plsc

_N = 1048576
_VOCAB = 10
_D = 16
_NC = 2
_NS = 16
_NW = _NC * _NS
_CHUNK = _N // _NW
_LANES = 16
_VECS = _CHUNK // _LANES

_M3 = 0x71C71C7
_INNER = 7
_CHAINS = 4
_BLOCK = _INNER * _CHAINS * 2
_OUTER = _VECS // _BLOCK
_TAIL = _VECS - _OUTER * _BLOCK
_SEG1 = _OUTER // 2
_SPLIT = _SEG1 * _BLOCK * _LANES


def _build_partial_call():
  mesh = plsc.VectorSubcoreMesh(
      core_axis_name="c", subcore_axis_name="s", num_cores=_NC)

  @functools.partial(
      pl.kernel,
      mesh=mesh,
      compiler_params=pltpu.CompilerParams(needs_layout_passes=False),
      out_type=jax.ShapeDtypeStruct((_NW, _D), jnp.float32),
      scratch_types=[
          pltpu.VMEM((_CHUNK,), jnp.int32),
          pltpu.VMEM((_VOCAB, _D), jnp.float32),
          pltpu.VMEM((_D,), jnp.float32),
          pltpu.VMEM((_LANES * _LANES,), jnp.int32),
          pltpu.VMEM((_LANES * _LANES,), jnp.int32),
          pltpu.SemaphoreType.DMA,
          pltpu.SemaphoreType.DMA,
      ],
  )
  def digit_hist_partial(digits_hbm, table_hbm, out_hbm,
                         chunk_v, table_v, row_v, cnt_v, rep_v,
                         sem1, sem2):
    c = lax.axis_index("c")
    s = lax.axis_index("s")
    wid = c * _NS + s
    base = wid * _CHUNK

    cp1 = pltpu.async_copy(
        digits_hbm.at[pl.ds(base, _SPLIT)], chunk_v.at[pl.ds(0, _SPLIT)], sem1)
    cp2 = pltpu.async_copy(
        digits_hbm.at[pl.ds(base + _SPLIT, _CHUNK - _SPLIT)],
        chunk_v.at[pl.ds(_SPLIT, _CHUNK - _SPLIT)], sem2)
    pltpu.sync_copy(table_hbm, table_v)

    def inner_block(vbase, n, ef):
      e, f = ef
      accs = [jnp.zeros((_LANES,), jnp.int32) for _ in range(_CHAINS)]
      for a in range(_CHAINS):
        for j in range(n):
          vec = chunk_v[pl.ds((vbase + a * n + j) * _LANES, _LANES)]
          accs[a] = accs[a] + (jnp.int32(1) << (vec * 3))
      for a in range(_CHAINS):
        e = e + (accs[a] & _M3)
        f = f + ((accs[a] >> 3) & _M3)
      return e, f

    def mid_flush(ef, cnts):
      e, f = ef
      cnts = list(cnts)
      for k in range(5):
        cnts[2 * k] = cnts[2 * k] + ((e >> (6 * k)) & 63)
        cnts[2 * k + 1] = cnts[2 * k + 1] + ((f >> (6 * k)) & 63)
      return tuple(cnts)

    def outer_body(i, cnts):
      ef = (jnp.zeros((_LANES,), jnp.int32), jnp.zeros((_LANES,), jnp.int32))
      vbase = i * _BLOCK
      ef = inner_block(vbase, _INNER, ef)
      ef = inner_block(vbase + _INNER * _CHAINS, _INNER, ef)
      return mid_flush(ef, cnts)

    cnts0 = tuple(jnp.zeros((_LANES,), jnp.int32) for _ in range(_VOCAB))
    cp1.wait()
    cnts = lax.fori_loop(0, _SEG1, outer_body, cnts0)
    cp2.wait()
    cnts = lax.fori_loop(_SEG1, _OUTER, outer_body, cnts)
    ef = (jnp.zeros((_LANES,), jnp.int32), jnp.zeros((_LANES,), jnp.int32))
    vb = _OUTER * _BLOCK
    q = _TAIL // _CHAINS
    while q > 0:
      n = min(q, _INNER)
      ef = inner_block(vb, n, ef)
      vb += n * _CHAINS
      q -= n
    accs = mid_flush(ef, cnts)

    for d in range(_VOCAB):
      cnt_v[pl.ds(d * _LANES, _LANES)] = accs[d]
    lane = lax.iota(jnp.int32, _LANES)
    totals = jnp.zeros((_LANES,), jnp.int32)
    for l in range(_LANES):
      totals = totals + plsc.load_gather(cnt_v, [lane * _LANES + l])
    for l in range(_LANES):
      plsc.store_scatter(rep_v, [lane * _LANES + l], totals)

    partial = jnp.zeros((_D,), jnp.float32)
    for d in range(_VOCAB):
      splat = rep_v[pl.ds(d * _LANES, _LANES)]
      partial = partial + splat.astype(jnp.float32) * table_v[d, :]
    row_v[...] = partial * jnp.float32(1.0 / _N)
    pltpu.sync_copy(row_v, out_hbm.at[wid])

  return digit_hist_partial


def _build_tc_fold():
  def fold_body(parts_ref, out_ref):
    out_ref[...] = jnp.sum(parts_ref[...], axis=0, keepdims=True)

  return pl.pallas_call(
      fold_body,
      out_shape=jax.ShapeDtypeStruct((1, _D), jnp.float32),
  )


_partial_call = _build_partial_call()
_tc_fold = _build_tc_fold()


def kernel(digits, table):
  digits = digits.reshape(_N).astype(jnp.int32)
  table = table.astype(jnp.float32)
  partials = _partial_call(digits, table)
  return _tc_fold(partials).reshape(_D)


# speedup vs baseline: 1.0540x; 1.0026x over previous
import functools

import jax
import jax.numpy as jnp
from jax import lax
from jax.experimental import pallas as pl
from jax.experimental.pallas import tpu as pltpu
from jax.experimental.pallas import tpu_sc as plsc

_N = 1048576
_VOCAB = 10
_D = 16
_NC = 2
_NS = 16
_NW = _NC * _NS
_CHUNK = _N // _NW
_LANES = 16
_VECS = _CHUNK // _LANES

_M3 = 0x71C71C7
_INNER = 7
_CHAINS = 4
_BLOCK = _INNER * _CHAINS * 2
_OUTER = _VECS // _BLOCK
_TAIL = _VECS - _OUTER * _BLOCK
_SEG1 = _OUTER // 2
_SPLIT = _SEG1 * _BLOCK * _LANES


def _build_partial_call():
  mesh = plsc.VectorSubcoreMesh(
      core_axis_name="c", subcore_axis_name="s", num_cores=_NC)

  @functools.partial(
      pl.kernel,
      mesh=mesh,
      compiler_params=pltpu.CompilerParams(needs_layout_passes=False),
      out_type=jax.ShapeDtypeStruct((_NW, _D), jnp.float32),
      scratch_types=[
          pltpu.VMEM((_CHUNK,), jnp.int32),
          pltpu.VMEM((_VOCAB, _D), jnp.float32),
          pltpu.VMEM((_D,), jnp.float32),
          pltpu.VMEM((_LANES * _LANES,), jnp.int32),
          pltpu.VMEM((_LANES * _LANES,), jnp.int32),
          pltpu.SemaphoreType.DMA,
          pltpu.SemaphoreType.DMA,
      ],
  )
  def digit_hist_partial(digits_hbm, table_hbm, out_hbm,
                         chunk_v, table_v, row_v, cnt_v, rep_v,
                         sem1, sem2):
    c = lax.axis_index("c")
    s = lax.axis_index("s")
    wid = c * _NS + s
    base = wid * _CHUNK

    cp1 = pltpu.async_copy(
        digits_hbm.at[pl.ds(base, _SPLIT)], chunk_v.at[pl.ds(0, _SPLIT)], sem1)
    cp2 = pltpu.async_copy(
        digits_hbm.at[pl.ds(base + _SPLIT, _CHUNK - _SPLIT)],
        chunk_v.at[pl.ds(_SPLIT, _CHUNK - _SPLIT)], sem2)
    pltpu.sync_copy(table_hbm, table_v)

    def inner_block(vbase, n, ef):
      e, f = ef
      accs = [jnp.zeros((_LANES,), jnp.int32) for _ in range(_CHAINS)]
      for a in range(_CHAINS):
        for j in range(n):
          vec = chunk_v[pl.ds((vbase + a * n + j) * _LANES, _LANES)]
          accs[a] = accs[a] + (jnp.int32(1) << (vec * 3))
      for a in range(_CHAINS):
        e = e + (accs[a] & _M3)
        f = f + ((accs[a] >> 3) & _M3)
      return e, f

    def mid_flush(ef, cnts):
      e, f = ef
      cnts = list(cnts)
      for k in range(5):
        cnts[2 * k] = cnts[2 * k] + ((e >> (6 * k)) & 63)
        cnts[2 * k + 1] = cnts[2 * k + 1] + ((f >> (6 * k)) & 63)
      return tuple(cnts)

    def outer_body(i, cnts):
      ef = (jnp.zeros((_LANES,), jnp.int32), jnp.zeros((_LANES,), jnp.int32))
      vbase = i * _BLOCK
      ef = inner_block(vbase, _INNER, ef)
      ef = inner_block(vbase + _INNER * _CHAINS, _INNER, ef)
      return mid_flush(ef, cnts)

    cnts0 = tuple(jnp.zeros((_LANES,), jnp.int32) for _ in range(_VOCAB))
    cp1.wait()
    cnts = lax.fori_loop(0, _SEG1, outer_body, cnts0)
    cp2.wait()
    cnts = lax.fori_loop(_SEG1, _OUTER, outer_body, cnts)
    ef = (jnp.zeros((_LANES,), jnp.int32), jnp.zeros((_LANES,), jnp.int32))
    vb = _OUTER * _BLOCK
    q = _TAIL // _CHAINS
    while q > 0:
      n = min(q, _INNER)
      ef = inner_block(vb, n, ef)
      vb += n * _CHAINS
      q -= n
    accs = mid_flush(ef, cnts)

    for d in range(_VOCAB):
      cnt_v[pl.ds(d * _LANES, _LANES)] = accs[d]
    zero = jnp.zeros((_LANES,), jnp.int32)
    for d in range(_VOCAB, _LANES):
      cnt_v[pl.ds(d * _LANES, _LANES)] = zero
    lane = lax.iota(jnp.int32, _LANES)
    totals = jnp.zeros((_LANES,), jnp.int32)
    for l in range(_LANES):
      totals = totals + plsc.load_gather(cnt_v, [lane * _LANES + l])
    for l in range(_LANES):
      plsc.store_scatter(rep_v, [lane * _LANES + l], totals)

    partial = jnp.zeros((_D,), jnp.float32)
    for d in range(_VOCAB):
      splat = rep_v[pl.ds(d * _LANES, _LANES)]
      partial = partial + splat.astype(jnp.float32) * table_v[d, :]
    row_v[...] = partial * jnp.float32(1.0 / _N)
    pltpu.sync_copy(row_v, out_hbm.at[wid])

  return digit_hist_partial


def _build_tc_fold():
  def fold_body(parts_ref, out_ref):
    out_ref[...] = jnp.sum(parts_ref[...], axis=0, keepdims=True)

  return pl.pallas_call(
      fold_body,
      out_shape=jax.ShapeDtypeStruct((1, _D), jnp.float32),
  )


_partial_call = _build_partial_call()
_tc_fold = _build_tc_fold()


def kernel(digits, table):
  digits = digits.reshape(_N).astype(jnp.int32)
  table = table.astype(jnp.float32)
  partials = _partial_call(digits, table)
  return _tc_fold(partials).reshape(_D)
